# baseline (speedup 1.0000x reference)
import functools

import jax
import jax.numpy as jnp
from jax import lax
from jax.experimental import pallas as pl
from jax.experimental.pallas import tpu as pltpu
from jax.experimental.pallas import tpu_sc as plsc

MAX_PT = 1000000
EMBED_DIM = 32
BATCH = 16384

NC = 2
NS = 16
NW = NC * NS
B_PER_W = BATCH // NW
CHUNK = 128
N_CHUNK = B_PER_W // CHUNK

TCOL = 128
NFULL = MAX_PT // TCOL
TAIL = MAX_PT - NFULL * TCOL
TAIL_TILE = 4
BLK_ELEMS = TCOL * EMBED_DIM

_mesh = plsc.VectorSubcoreMesh(core_axis_name="c", subcore_axis_name="s")


@functools.partial(
    pl.kernel,
    mesh=_mesh,
    out_type=jax.ShapeDtypeStruct((MAX_PT * EMBED_DIM,), jnp.float32),
    compiler_params=pltpu.CompilerParams(needs_layout_passes=False),
    scratch_types=[
        pltpu.VMEM((EMBED_DIM, TCOL), jnp.float32),
        pltpu.VMEM((BLK_ELEMS,), jnp.float32),
        pltpu.SemaphoreType.DMA,
    ],
)
def _transpose_kernel(table_hbm, out_hbm, blk_v, out_v, sem):
    wid = lax.axis_index("s") * NC + lax.axis_index("c")
    nblk = lax.select(wid < NFULL % NW, NFULL // NW + 1, NFULL // NW)
    lanes = lax.iota(jnp.int32, 16)

    def body(b, carry):
        ti = wid + b * NW
        pltpu.sync_copy(table_hbm.at[:, pl.ds(ti * TCOL, TCOL)], blk_v)
        for j in range(EMBED_DIM):
            idx_j = lanes * EMBED_DIM + j
            for i0 in range(TCOL // 16):
                v = blk_v[j, pl.ds(i0 * 16, 16)]
                plsc.store_scatter(
                    out_v.at[pl.ds(i0 * 16 * EMBED_DIM, 16 * EMBED_DIM)],
                    [idx_j], v)
        pltpu.sync_copy(out_v, out_hbm.at[pl.ds(ti * BLK_ELEMS, BLK_ELEMS)])
        return carry

    lax.fori_loop(0, nblk, body, 0)


@functools.partial(
    pl.kernel,
    mesh=_mesh,
    out_type=jax.ShapeDtypeStruct((BATCH, EMBED_DIM), jnp.float32),
    compiler_params=pltpu.CompilerParams(use_tc_tiling_on_sc=False),
    scratch_types=[
        pltpu.VMEM((N_CHUNK, CHUNK), jnp.int32),
        pltpu.VMEM((B_PER_W, EMBED_DIM), jnp.float32),
        pltpu.SemaphoreType.DMA,
    ],
)
def _gather_kernel(table_hbm, idx_hbm, out_hbm, idx_v, rows_v, sem):
    wid = lax.axis_index("s") * NC + lax.axis_index("c")
    base = wid * B_PER_W
    pltpu.sync_copy(idx_hbm.at[wid], idx_v)
    copies = []
    for j in range(N_CHUNK):
        copies.append(
            pltpu.async_copy(
                table_hbm.at[idx_v.at[j]],
                rows_v.at[pl.ds(j * CHUNK, CHUNK), :],
                sem,
            )
        )
    for c in copies:
        c.wait()
    pltpu.sync_copy(rows_v, out_hbm.at[pl.ds(base, B_PER_W)])


def kernel(pt_id, theta_h_weight):
    tw_lin = _transpose_kernel(theta_h_weight.T).reshape(MAX_PT, EMBED_DIM)
    idx = jnp.clip(pt_id.astype(jnp.int32), 0, MAX_PT - 1)
    main = _gather_kernel(tw_lin, idx.reshape(NW, N_CHUNK, CHUNK))
    tail_tbl = theta_h_weight[NFULL * TCOL:, :]
    is_tail = idx >= NFULL * TCOL
    t = jnp.clip(idx - NFULL * TCOL, 0, TAIL - 1)
    onehot = ((t[:, None] == jnp.arange(TAIL, dtype=jnp.int32)[None, :])
              & is_tail[:, None]).astype(jnp.float32)
    tail_vals = onehot @ tail_tbl
    return jnp.where(is_tail[:, None], tail_vals, main)


# speedup vs baseline: 1.3574x; 1.3574x over previous
import functools

import jax
import jax.numpy as jnp
from jax import lax
from jax.experimental import pallas as pl
from jax.experimental.pallas import tpu as pltpu
from jax.experimental.pallas import tpu_sc as plsc

MAX_PT = 1000000
EMBED_DIM = 32
BATCH = 16384

NC = 2
NS = 16
NW = NC * NS
B_PER_W = BATCH // NW
CHUNK = 128
N_CHUNK = B_PER_W // CHUNK

TCOL = 512
NBLK = MAX_PT // TCOL
TAIL = MAX_PT - NBLK * TCOL
BLK_ELEMS = TCOL * EMBED_DIM

_mesh = plsc.VectorSubcoreMesh(core_axis_name="c", subcore_axis_name="s")


@functools.partial(
    pl.kernel,
    mesh=_mesh,
    out_type=jax.ShapeDtypeStruct((MAX_PT * EMBED_DIM,), jnp.float32),
    compiler_params=pltpu.CompilerParams(needs_layout_passes=False),
    scratch_types=[
        pltpu.VMEM((2 * EMBED_DIM, TCOL), jnp.float32),
        pltpu.VMEM((2 * BLK_ELEMS,), jnp.float32),
        pltpu.SemaphoreType.DMA,
        pltpu.SemaphoreType.DMA,
    ],
)
def _transpose_kernel(table_hbm, out_hbm, blk_v, out_v, sem_in, sem_out):
    wid = lax.axis_index("s") * NC + lax.axis_index("c")
    nblk = lax.select(wid < NBLK % NW, NBLK // NW + 1, NBLK // NW)
    lanes = lax.iota(jnp.int32, 16)

    def in_src(b):
        return table_hbm.at[:, pl.ds((wid + b * NW) * TCOL, TCOL)]

    def out_dst(b):
        return out_hbm.at[pl.ds((wid + b * NW) * BLK_ELEMS, BLK_ELEMS)]

    def in_buf(p):
        return blk_v.at[pl.ds(p * EMBED_DIM, EMBED_DIM), :]

    def out_buf(p):
        return out_v.at[pl.ds(p * BLK_ELEMS, BLK_ELEMS)]

    pltpu.async_copy(in_src(0), in_buf(0), sem_in)

    def body(b, carry):
        p = b % 2

        @pl.when(b + 1 < nblk)
        def _fire_next():
            pltpu.async_copy(in_src(b + 1), in_buf((b + 1) % 2), sem_in)

        pltpu.make_async_copy(in_src(b), in_buf(p), sem_in).wait()

        @pl.when(b >= 2)
        def _drain_out():
            pltpu.make_async_copy(out_buf(p), out_dst(b - 2), sem_out).wait()

        for j in range(EMBED_DIM):
            idx_j = lanes * EMBED_DIM + j
            row = p * EMBED_DIM + j
            for i0 in range(TCOL // 16):
                v = blk_v[row, pl.ds(i0 * 16, 16)]
                plsc.store_scatter(
                    out_v.at[pl.ds(p * BLK_ELEMS + i0 * 16 * EMBED_DIM,
                                   16 * EMBED_DIM)],
                    [idx_j], v)

        pltpu.async_copy(out_buf(p), out_dst(b), sem_out)
        return carry

    lax.fori_loop(0, nblk, body, 0)

    @pl.when(nblk >= 2)
    def _drain_tail2():
        pltpu.make_async_copy(out_buf(0), out_dst(nblk - 2), sem_out).wait()

    pltpu.make_async_copy(out_buf(0), out_dst(nblk - 1), sem_out).wait()


@functools.partial(
    pl.kernel,
    mesh=_mesh,
    out_type=jax.ShapeDtypeStruct((BATCH, EMBED_DIM), jnp.float32),
    compiler_params=pltpu.CompilerParams(use_tc_tiling_on_sc=False),
    scratch_types=[
        pltpu.VMEM((N_CHUNK, CHUNK), jnp.int32),
        pltpu.VMEM((B_PER_W, EMBED_DIM), jnp.float32),
        pltpu.SemaphoreType.DMA,
    ],
)
def _gather_kernel(table_hbm, idx_hbm, out_hbm, idx_v, rows_v, sem):
    wid = lax.axis_index("s") * NC + lax.axis_index("c")
    base = wid * B_PER_W
    pltpu.sync_copy(idx_hbm.at[wid], idx_v)
    copies = []
    for j in range(N_CHUNK):
        copies.append(
            pltpu.async_copy(
                table_hbm.at[idx_v.at[j]],
                rows_v.at[pl.ds(j * CHUNK, CHUNK), :],
                sem,
            )
        )
    for c in copies:
        c.wait()
    pltpu.sync_copy(rows_v, out_hbm.at[pl.ds(base, B_PER_W)])


def kernel(pt_id, theta_h_weight):
    tw_lin = _transpose_kernel(theta_h_weight.T).reshape(MAX_PT, EMBED_DIM)
    idx = jnp.clip(pt_id.astype(jnp.int32), 0, MAX_PT - 1)
    main = _gather_kernel(tw_lin, idx.reshape(NW, N_CHUNK, CHUNK))
    tail_tbl = theta_h_weight[NBLK * TCOL:, :]
    is_tail = idx >= NBLK * TCOL
    t = jnp.clip(idx - NBLK * TCOL, 0, TAIL - 1)
    onehot = ((t[:, None] == jnp.arange(TAIL, dtype=jnp.int32)[None, :])
              & is_tail[:, None]).astype(jnp.float32)
    tail_vals = onehot @ tail_tbl
    return jnp.where(is_tail[:, None], tail_vals, main)


# speedup vs baseline: 1.3679x; 1.0078x over previous
import functools

import jax
import jax.numpy as jnp
from jax import lax
from jax.experimental import pallas as pl
from jax.experimental.pallas import tpu as pltpu
from jax.experimental.pallas import tpu_sc as plsc

MAX_PT = 1000000
EMBED_DIM = 32
BATCH = 16384

NC = 2
NS = 16
NW = NC * NS
B_PER_W = BATCH // NW
CHUNK = 128
N_CHUNK = B_PER_W // CHUNK

TCOL = 512
NBLK = MAX_PT // TCOL
TAIL = MAX_PT - NBLK * TCOL
BLK_ELEMS = TCOL * EMBED_DIM

_mesh = plsc.VectorSubcoreMesh(core_axis_name="c", subcore_axis_name="s")


@functools.partial(
    pl.kernel,
    mesh=_mesh,
    out_type=jax.ShapeDtypeStruct((MAX_PT * EMBED_DIM,), jnp.float32),
    compiler_params=pltpu.CompilerParams(needs_layout_passes=False),
    scratch_types=[
        pltpu.VMEM((2 * EMBED_DIM, TCOL), jnp.float32),
        pltpu.VMEM((2 * BLK_ELEMS,), jnp.float32),
        pltpu.SemaphoreType.DMA,
        pltpu.SemaphoreType.DMA,
    ],
)
def _transpose_kernel(table_hbm, out_hbm, blk_v, out_v, sem_in, sem_out):
    wid = lax.axis_index("s") * NC + lax.axis_index("c")
    nblk = lax.select(wid < NBLK % NW, NBLK // NW + 1, NBLK // NW)
    lanes = lax.iota(jnp.int32, 16)

    def in_src(b):
        return table_hbm.at[:, pl.ds((wid + b * NW) * TCOL, TCOL)]

    def out_dst(b):
        return out_hbm.at[pl.ds((wid + b * NW) * BLK_ELEMS, BLK_ELEMS)]

    def in_buf(p):
        return blk_v.at[pl.ds(p * EMBED_DIM, EMBED_DIM), :]

    def out_buf(p):
        return out_v.at[pl.ds(p * BLK_ELEMS, BLK_ELEMS)]

    pltpu.async_copy(in_src(0), in_buf(0), sem_in)

    def body(b, carry):
        p = b % 2

        @pl.when(b + 1 < nblk)
        def _fire_next():
            pltpu.async_copy(in_src(b + 1), in_buf((b + 1) % 2), sem_in)

        pltpu.make_async_copy(in_src(b), in_buf(p), sem_in).wait()

        @pl.when(b >= 2)
        def _drain_out():
            pltpu.make_async_copy(out_buf(p), out_dst(b - 2), sem_out).wait()

        win = out_v.at[pl.ds(p * BLK_ELEMS, BLK_ELEMS)]
        for j in range(EMBED_DIM):
            row = p * EMBED_DIM + j
            idx = lanes * EMBED_DIM + j
            for i0 in range(TCOL // 16):
                v = blk_v[row, pl.ds(i0 * 16, 16)]
                plsc.store_scatter(win, [idx], v)
                idx = idx + 16 * EMBED_DIM

        pltpu.async_copy(out_buf(p), out_dst(b), sem_out)
        return carry

    lax.fori_loop(0, nblk, body, 0)

    @pl.when(nblk >= 2)
    def _drain_tail2():
        pltpu.make_async_copy(out_buf(0), out_dst(nblk - 2), sem_out).wait()

    pltpu.make_async_copy(out_buf(0), out_dst(nblk - 1), sem_out).wait()


@functools.partial(
    pl.kernel,
    mesh=_mesh,
    out_type=jax.ShapeDtypeStruct((BATCH, EMBED_DIM), jnp.float32),
    compiler_params=pltpu.CompilerParams(use_tc_tiling_on_sc=False),
    scratch_types=[
        pltpu.VMEM((N_CHUNK, CHUNK), jnp.int32),
        pltpu.VMEM((B_PER_W, EMBED_DIM), jnp.float32),
        pltpu.SemaphoreType.DMA,
    ],
)
def _gather_kernel(table_hbm, idx_hbm, out_hbm, idx_v, rows_v, sem):
    wid = lax.axis_index("s") * NC + lax.axis_index("c")
    base = wid * B_PER_W
    pltpu.sync_copy(idx_hbm.at[wid], idx_v)
    copies = []
    for j in range(N_CHUNK):
        copies.append(
            pltpu.async_copy(
                table_hbm.at[idx_v.at[j]],
                rows_v.at[pl.ds(j * CHUNK, CHUNK), :],
                sem,
            )
        )
    for c in copies:
        c.wait()
    pltpu.sync_copy(rows_v, out_hbm.at[pl.ds(base, B_PER_W)])


def kernel(pt_id, theta_h_weight):
    tw_lin = _transpose_kernel(theta_h_weight.T).reshape(MAX_PT, EMBED_DIM)
    idx = jnp.clip(pt_id.astype(jnp.int32), 0, MAX_PT - 1)
    main = _gather_kernel(tw_lin, idx.reshape(NW, N_CHUNK, CHUNK))
    tail_tbl = theta_h_weight[NBLK * TCOL:, :]
    is_tail = idx >= NBLK * TCOL
    t = jnp.clip(idx - NBLK * TCOL, 0, TAIL - 1)
    onehot = ((t[:, None] == jnp.arange(TAIL, dtype=jnp.int32)[None, :])
              & is_tail[:, None]).astype(jnp.float32)
    tail_vals = onehot @ tail_tbl
    return jnp.where(is_tail[:, None], tail_vals, main)
